# baseline (speedup 1.0000x reference)
import jax
import jax.numpy as jnp
from jax import lax
from jax.experimental import pallas as pl
from jax.experimental.pallas import tpu as pltpu
from jax.experimental.pallas import tpu_sc as plsc

_DD = 1000000
_NC = 2
_NS = 16
_NW = _NC * _NS
_B = 512
_NCH = _B // 128
_Z0 = 16384
_Z1 = 200
_F = 32
_TPJ = _Z0 // _B
_JB = 4
_ZB = 2048


def _sc_body(z_hbm, u_hbm, out_hbm, z_v, idx0_v, idx1_v, w_v, f0_v, f1_v,
             sem_z, sem_g, sem_o0, sem_o1):
    wid = lax.axis_index("c") * _NS + lax.axis_index("s")
    nblk = (_Z1 * _TPJ) // _NW
    blk0 = wid * nblk
    sem_o = (sem_o0, sem_o1)

    def compute_idx(slot):
        def idx_body(g, c):
            o = g * 16
            z = z_v[slot, pl.ds(o, 16)]
            z = jnp.minimum(jnp.maximum(z, 0.0), 1.0)
            zi = z * jnp.float32(_DD - 1)
            z0 = zi.astype(jnp.int32)
            w = zi - z0.astype(jnp.float32)
            idx0_v[slot, pl.ds(o, 16)] = z0
            idx1_v[slot, pl.ds(o, 16)] = jnp.minimum(z0 + 1, _DD - 1)
            w_v[slot, pl.ds(o, 16)] = w
            return c
        lax.fori_loop(0, _B // 16, idx_body, 0)

    def gather_copies(slot):
        copies = []
        for c in range(_NCH):
            o = c * 128
            copies.append(pltpu.make_async_copy(
                u_hbm.at[idx0_v.at[slot, pl.ds(o, 128)]],
                f0_v.at[slot, pl.ds(o, 128)], sem_g))
            copies.append(pltpu.make_async_copy(
                u_hbm.at[idx1_v.at[slot, pl.ds(o, 128)]],
                f1_v.at[slot, pl.ds(o, 128)], sem_g))
        return copies

    def fire_gathers(slot):
        for cp in gather_copies(slot):
            cp.start()

    def out_copy(slot, b):
        return pltpu.make_async_copy(
            f0_v.at[slot],
            out_hbm.at[pl.ds(b * _B, _B)], sem_o[slot])

    def mix(slot):
        def mix_body(g, c):
            wg = w_v[slot, pl.ds(g * 16, 16)]
            i0 = g * 16
            for l in range(16):
                i = i0 + l
                w = lax.broadcast_in_dim(
                    lax.slice(wg, (l,), (l + 1,)), (16,), (0,))
                a0 = f0_v[slot, i, pl.ds(0, 16)]
                b0 = f1_v[slot, i, pl.ds(0, 16)]
                a1 = f0_v[slot, i, pl.ds(16, 16)]
                b1 = f1_v[slot, i, pl.ds(16, 16)]
                f0_v[slot, i, pl.ds(0, 16)] = a0 + w * (b0 - a0)
                f0_v[slot, i, pl.ds(16, 16)] = a1 + w * (b1 - a1)
            return c
        lax.fori_loop(0, _B // 16, mix_body, 0)

    pltpu.sync_copy(z_hbm.at[pl.ds(blk0 * _B, _B)], z_v.at[0])
    compute_idx(0)
    fire_gathers(0)

    def blk_body(k, carry):
        p = lax.rem(k, 2)
        b = blk0 + k

        def half(p, q):
            zcp = pltpu.make_async_copy(
                z_hbm.at[pl.ds((b + 1) * _B, _B)], z_v.at[q], sem_z)

            @pl.when(k + 1 < nblk)
            def _():
                zcp.start()

            for cp in gather_copies(p):
                cp.wait()

            @pl.when(k + 1 < nblk)
            def _():
                zcp.wait()
                compute_idx(q)
                fire_gathers(q)

            @pl.when(k >= 2)
            def _():
                out_copy(p, b - 2).wait()

            mix(p)
            out_copy(p, b).start()

        @pl.when(p == 0)
        def _():
            half(0, 1)

        @pl.when(p == 1)
        def _():
            half(1, 0)

        return carry

    lax.fori_loop(0, nblk, blk_body, 0)

    out_copy((nblk - 2) % 2, blk0 + nblk - 2).wait()
    out_copy((nblk - 1) % 2, blk0 + nblk - 1).wait()


def _tc_transpose_body(x_ref, o_ref):
    x = x_ref[...]
    s = _ZB // 128
    x6 = x.reshape(_JB, s, 32, 4, 4, 8)
    y = x6.transpose(0, 4, 1, 5, 2, 3)
    o_ref[...] = y.reshape(_JB, 4, s, 8, 128)


def kernel(z_grid, U):
    n = z_grid.shape[0] * z_grid.shape[1]
    zt_flat = z_grid.T.reshape(n)
    mesh = plsc.VectorSubcoreMesh(
        core_axis_name="c", subcore_axis_name="s",
        num_cores=_NC, num_subcores=_NS)
    sc_kern = pl.kernel(
        _sc_body,
        out_type=jax.ShapeDtypeStruct((n, _F), jnp.float32),
        mesh=mesh,
        scratch_types=[
            pltpu.VMEM((2, _B), jnp.float32),
            pltpu.VMEM((2, _B), jnp.int32),
            pltpu.VMEM((2, _B), jnp.int32),
            pltpu.VMEM((2, _B), jnp.float32),
            pltpu.VMEM((2, _B, _F), jnp.float32),
            pltpu.VMEM((2, _B, _F), jnp.float32),
            pltpu.SemaphoreType.DMA,
            pltpu.SemaphoreType.DMA,
            pltpu.SemaphoreType.DMA,
            pltpu.SemaphoreType.DMA,
        ],
        compiler_params=pltpu.CompilerParams(use_tc_tiling_on_sc=False),
    )
    rows = sc_kern(zt_flat, U)

    z0, z1 = z_grid.shape
    x3 = rows.reshape(z1, z0 * _F // 128, 128)
    out5 = pl.pallas_call(
        _tc_transpose_body,
        grid=(z1 // _JB, z0 // _ZB),
        in_specs=[pl.BlockSpec(
            (_JB, _ZB * _F // 128, 128), lambda jb, zb: (jb, zb, 0))],
        out_specs=pl.BlockSpec(
            (_JB, _F // 8, _ZB // 128, 8, 128),
            lambda jb, zb: (jb, 0, zb, 0, 0)),
        out_shape=jax.ShapeDtypeStruct(
            (z1, _F // 8, z0 // 128, 8, 128), jnp.float32),
    )(x3)
    return out5.transpose(2, 4, 0, 1, 3).reshape(z0, z1, _F)


# speedup vs baseline: 8.7617x; 8.7617x over previous
import jax
import jax.numpy as jnp
from jax import lax
from jax.experimental import pallas as pl
from jax.experimental.pallas import tpu as pltpu
from jax.experimental.pallas import tpu_sc as plsc

_DD = 1000000
_NC = 2
_NS = 16
_NW = _NC * _NS
_B = 512
_NCH = _B // 128
_Z0 = 16384
_Z1 = 200
_F = 32
_TPJ = _Z0 // _B


def _sc_body(z_hbm, u_hbm, out_hbm, z_v, idx0_v, idx1_v, w_v, f0_v, f1_v,
             sem_z, sem_g, sem_o0, sem_o1):
    wid = lax.axis_index("c") * _NS + lax.axis_index("s")
    nblk = (_Z1 * _TPJ) // _NW
    blk0 = wid * nblk
    sem_o = (sem_o0, sem_o1)

    def compute_idx(slot):
        def idx_body(g, c):
            o = g * 16
            z = z_v[slot, pl.ds(o, 16)]
            z = jnp.minimum(jnp.maximum(z, 0.0), 1.0)
            zi = z * jnp.float32(_DD - 1)
            z0 = zi.astype(jnp.int32)
            w = zi - z0.astype(jnp.float32)
            idx0_v[slot, pl.ds(o, 16)] = z0
            idx1_v[slot, pl.ds(o, 16)] = jnp.minimum(z0 + 1, _DD - 1)
            w_v[slot, pl.ds(o, 16)] = w
            return c
        lax.fori_loop(0, _B // 16, idx_body, 0)

    def gather_copies(slot):
        copies = []
        for c in range(_NCH):
            o = c * 128
            copies.append(pltpu.make_async_copy(
                u_hbm.at[idx0_v.at[slot, pl.ds(o, 128)]],
                f0_v.at[slot, pl.ds(o, 128)], sem_g))
            copies.append(pltpu.make_async_copy(
                u_hbm.at[idx1_v.at[slot, pl.ds(o, 128)]],
                f1_v.at[slot, pl.ds(o, 128)], sem_g))
        return copies

    def fire_gathers(slot):
        for cp in gather_copies(slot):
            cp.start()

    def out_copy(slot, b):
        return pltpu.make_async_copy(
            f0_v.at[slot],
            out_hbm.at[pl.ds(b * _B, _B)], sem_o[slot])

    def mix(slot):
        def mix_body(g, c):
            wg = w_v[slot, pl.ds(g * 16, 16)]
            i0 = g * 16
            for l in range(16):
                i = i0 + l
                w = lax.broadcast_in_dim(
                    lax.slice(wg, (l,), (l + 1,)), (16,), (0,))
                a0 = f0_v[slot, i, pl.ds(0, 16)]
                b0 = f1_v[slot, i, pl.ds(0, 16)]
                a1 = f0_v[slot, i, pl.ds(16, 16)]
                b1 = f1_v[slot, i, pl.ds(16, 16)]
                f0_v[slot, i, pl.ds(0, 16)] = a0 + w * (b0 - a0)
                f0_v[slot, i, pl.ds(16, 16)] = a1 + w * (b1 - a1)
            return c
        lax.fori_loop(0, _B // 16, mix_body, 0)

    pltpu.sync_copy(z_hbm.at[pl.ds(blk0 * _B, _B)], z_v.at[0])
    compute_idx(0)
    fire_gathers(0)

    def blk_body(k, carry):
        p = lax.rem(k, 2)
        b = blk0 + k

        def half(p, q):
            zcp = pltpu.make_async_copy(
                z_hbm.at[pl.ds((b + 1) * _B, _B)], z_v.at[q], sem_z)

            @pl.when(k + 1 < nblk)
            def _():
                zcp.start()

            for cp in gather_copies(p):
                cp.wait()

            @pl.when(k + 1 < nblk)
            def _():
                zcp.wait()
                compute_idx(q)
                fire_gathers(q)

            @pl.when(k >= 2)
            def _():
                out_copy(p, b - 2).wait()

            mix(p)
            out_copy(p, b).start()

        @pl.when(p == 0)
        def _():
            half(0, 1)

        @pl.when(p == 1)
        def _():
            half(1, 0)

        return carry

    lax.fori_loop(0, nblk, blk_body, 0)

    out_copy((nblk - 2) % 2, blk0 + nblk - 2).wait()
    out_copy((nblk - 1) % 2, blk0 + nblk - 1).wait()


def kernel(z_grid, U):
    n = z_grid.shape[0] * z_grid.shape[1]
    zt_flat = z_grid.T.reshape(n)
    mesh = plsc.VectorSubcoreMesh(
        core_axis_name="c", subcore_axis_name="s",
        num_cores=_NC, num_subcores=_NS)
    sc_kern = pl.kernel(
        _sc_body,
        out_type=jax.ShapeDtypeStruct((n, _F), jnp.float32),
        mesh=mesh,
        scratch_types=[
            pltpu.VMEM((2, _B), jnp.float32),
            pltpu.VMEM((2, _B), jnp.int32),
            pltpu.VMEM((2, _B), jnp.int32),
            pltpu.VMEM((2, _B), jnp.float32),
            pltpu.VMEM((2, _B, _F), jnp.float32),
            pltpu.VMEM((2, _B, _F), jnp.float32),
            pltpu.SemaphoreType.DMA,
            pltpu.SemaphoreType.DMA,
            pltpu.SemaphoreType.DMA,
            pltpu.SemaphoreType.DMA,
        ],
        compiler_params=pltpu.CompilerParams(use_tc_tiling_on_sc=False),
    )
    rows = sc_kern(zt_flat, U)

    z0, z1 = z_grid.shape
    return rows.reshape(z1, z0, _F).transpose(1, 0, 2)
